# Initial kernel scaffold; baseline (speedup 1.0000x reference)
import jax
import jax.numpy as jnp
from jax.experimental import pallas as pl


def kernel(x, edge_index, embedding, W, att_i, att_j, att_em_i, att_em_j, bias, gamma, beta):
    raise NotImplementedError("write your pallas kernel here")


# speedup vs baseline: 5.0308x; 5.0308x over previous
import functools

import jax
import jax.numpy as jnp
from jax import lax
from jax.experimental import pallas as pl
from jax.experimental.pallas import tpu as pltpu
from jax.experimental.pallas import tpu_sc as plsc

N = 10000
E = 320000
CH = 128
NEG = 0.2

NPAD = 10240
NG = 16
EPG = 20640
NCH = EPG // 16
ETOT = NG * EPG
TRASH = N


def _prep_body(x_ref, w_ref, emb_ref, att_ref, xh_ref, aij_ref):
    xb = x_ref[...]
    xh = jnp.dot(xb, w_ref[...], preferred_element_type=jnp.float32)
    xh_ref[...] = xh
    t1 = jnp.dot(xh, att_ref[...].T, preferred_element_type=jnp.float32)
    t2 = jnp.dot(emb_ref[...], att_ref[...].T, preferred_element_type=jnp.float32)
    ai = t1[:, 0] + t2[:, 2]
    aj = t1[:, 1] + t2[:, 3]
    z = jnp.zeros((14, ai.shape[0]), jnp.float32)
    aij_ref[...] = jnp.concatenate([ai[None, :], aj[None, :], z], axis=0)


def _tc_prep(x_pad, W, emb_pad, attv):
    blk = 1024
    return pl.pallas_call(
        _prep_body,
        grid=(NPAD // blk,),
        in_specs=[
            pl.BlockSpec((blk, CH), lambda i: (i, 0)),
            pl.BlockSpec((CH, CH), lambda i: (0, 0)),
            pl.BlockSpec((blk, CH), lambda i: (i, 0)),
            pl.BlockSpec((8, CH), lambda i: (0, 0)),
        ],
        out_specs=[
            pl.BlockSpec((blk, CH), lambda i: (i, 0)),
            pl.BlockSpec((16, blk), lambda i: (0, i)),
        ],
        out_shape=[
            jax.ShapeDtypeStruct((NPAD, CH), jnp.float32),
            jax.ShapeDtypeStruct((16, NPAD), jnp.float32),
        ],
    )(x_pad, W, emb_pad, attv)


def _post_body(acc_ref, den_ref, bias_ref, gamma_ref, beta_ref, out_ref):
    a = jnp.concatenate([acc_ref[q, :N, :] for q in range(4)], axis=1)
    den = den_ref[0, :N]
    pre = a / (den[:, None] + 1e-16) + bias_ref[...]
    mean = jnp.mean(pre, axis=0, keepdims=True)
    dlt = pre - mean
    var = jnp.mean(dlt * dlt, axis=0, keepdims=True)
    o = dlt * (gamma_ref[...] / jnp.sqrt(var + 1e-5)) + beta_ref[...]
    out_ref[...] = jnp.maximum(o, 0.0)


def _tc_post(acc, dens, bias2, gamma2, beta2):
    return pl.pallas_call(
        _post_body,
        out_shape=jax.ShapeDtypeStruct((N, CH), jnp.float32),
    )(acc, dens, bias2, gamma2, beta2)


def _make_sc_edges():
    mesh = plsc.VectorSubcoreMesh(core_axis_name="c", subcore_axis_name="s")

    @functools.partial(
        pl.kernel,
        mesh=mesh,
        compiler_params=pltpu.CompilerParams(
            needs_layout_passes=False, use_tc_tiling_on_sc=False),
        out_type=[
            pltpu.HBM((4, NPAD, CH // 4), jnp.float32),
            pltpu.HBM((2, NPAD // 16, 16), jnp.float32),
        ],
        scratch_types=[
            pltpu.VMEM((NPAD,), jnp.float32),
            pltpu.VMEM((NPAD,), jnp.float32),
            pltpu.VMEM((NPAD // 16, 16), jnp.float32),
            pltpu.VMEM((NPAD // 16,), jnp.int32),
            pltpu.VMEM((NCH, 16), jnp.int32),
            pltpu.VMEM((NCH, 16), jnp.int32),
            pltpu.VMEM((NCH, 16), jnp.float32),
            pltpu.VMEM((16, CH // 4), jnp.float32),
            pltpu.VMEM((16, CH // 4), jnp.float32),
            pltpu.VMEM((128, CH // 4), jnp.float32),
            pltpu.VMEM_SHARED((NPAD, CH // 4), jnp.float32),
            pltpu.VMEM_SHARED((NPAD // 16, 16), jnp.float32),
            pltpu.SemaphoreType.DMA,
        ],
    )
    def sc_edges(xh_hbm, src_hbm, dst_hbm, aij_hbm, acc_out, den_out,
                 ai_v, aj_v, den_v, blk_v, src_v, dst_v, p_v, rows_v,
                 scaled_v, zer_v, acc_sh, den_sh, sem):
        c = lax.axis_index("c")
        s = lax.axis_index("s")

        pltpu.sync_copy(aij_hbm.at[0], ai_v)
        pltpu.sync_copy(aij_hbm.at[1], aj_v)
        pltpu.sync_copy(src_hbm.at[s], src_v)
        pltpu.sync_copy(dst_hbm.at[s], dst_v)

        z16 = jnp.zeros((16,), jnp.float32)
        iota16 = lax.iota(jnp.int32, 16)

        def zden(i, carry):
            den_v[i, :] = z16
            blk_v[pl.ds(i * 16, 16)] = iota16 + i * 16
            return carry
        lax.fori_loop(0, NPAD // 256, zden, 0)

        def zden2(i, carry):
            den_v[i, :] = z16
            return carry
        lax.fori_loop(NPAD // 256, NPAD // 16, zden2, 0)

        def zzer(i, carry):
            for k in range(2):
                zer_v[i, pl.ds(k * 16, 16)] = z16
            return carry
        lax.fori_loop(0, 128, zzer, 0)

        @pl.when(s == 0)
        def _():
            pltpu.sync_copy(den_v, den_sh)
        plsc.subcore_barrier()

        def ph1(j, carry):
            src16 = src_v[j, :]
            dst16 = dst_v[j, :]
            ai16 = plsc.load_gather(ai_v, [dst16])
            aj16 = plsc.load_gather(aj_v, [src16])
            al = ai16 + aj16
            al = jnp.where(al >= 0.0, al, NEG * al)
            p16 = jnp.exp(al)
            p_v[j, :] = p16
            plsc.addupdate_scatter(
                den_v, [lax.shift_right_logical(dst16, 4), dst16 & 15], p16)
            return carry
        lax.fori_loop(0, NCH, ph1, 0)

        pltpu.sync_copy(den_v, den_sh.at[blk_v], add=True)

        for q in range(2):
            coff = (c + 2 * q) * NPAD

            def zacc(m, carry):
                pltpu.sync_copy(zer_v, acc_sh.at[pl.ds(s * 640 + m * 128, 128)])
                return carry
            lax.fori_loop(0, 5, zacc, 0)
            plsc.subcore_barrier()

            def ph2(j, carry):
                idx16 = src_v[j, :] + coff
                pltpu.async_copy(xh_hbm.at[idx16], rows_v, sem).wait()
                jj = jnp.zeros((16,), jnp.int32) + j
                for r in range(16):
                    rr = jnp.full((16,), r, jnp.int32)
                    pv = plsc.load_gather(p_v, [jj, rr])
                    for k in range(2):
                        scaled_v[r, pl.ds(k * 16, 16)] = (
                            rows_v[r, pl.ds(k * 16, 16)] * pv)
                pltpu.sync_copy(scaled_v, acc_sh.at[dst_v.at[j]], add=True)
                return carry
            lax.fori_loop(0, NCH, ph2, 0)
            plsc.subcore_barrier()

            @pl.when(s == 0)
            def _():
                pltpu.sync_copy(acc_sh, acc_out.at[c + 2 * q])
            plsc.subcore_barrier()

        @pl.when(s == 0)
        def _():
            pltpu.sync_copy(den_sh, den_out.at[c])

    return sc_edges


def kernel(x, edge_index, embedding, W, att_i, att_j, att_em_i, att_em_j,
           bias, gamma, beta):
    src = edge_index[0]
    dst = edge_index[1]
    dst = jnp.where(src == dst, TRASH, dst)
    loop = jnp.arange(N, dtype=jnp.int32)
    pad = ETOT - (E + N)
    src_all = jnp.concatenate([src, loop, jnp.zeros((pad,), jnp.int32)])
    dst_all = jnp.concatenate([dst, loop, jnp.full((pad,), TRASH, jnp.int32)])
    src3 = src_all.reshape(NG, NCH, 16)
    dst3 = dst_all.reshape(NG, NCH, 16)

    x_pad = jnp.pad(x, ((0, NPAD - N), (0, 0)))
    emb_pad = jnp.pad(embedding, ((0, NPAD - N), (0, 0)))
    attv = jnp.concatenate(
        [att_i[0], att_j[0], att_em_i[0], att_em_j[0],
         jnp.zeros((4, CH), jnp.float32)], axis=0)

    xh, aij = _tc_prep(x_pad, W, emb_pad, attv)
    xh2 = xh.reshape(NPAD, 4, CH // 4).transpose(1, 0, 2).reshape(4 * NPAD, CH // 4)

    sc_edges = _make_sc_edges()
    acc, dens = sc_edges(xh2, src3, dst3, aij)

    return _tc_post(acc, dens.reshape(2, NPAD),
                    bias[None, :], gamma[None, :], beta[None, :])


# speedup vs baseline: 14.5006x; 2.8824x over previous
import functools

import jax
import jax.numpy as jnp
from jax import lax
from jax.experimental import pallas as pl
from jax.experimental.pallas import tpu as pltpu
from jax.experimental.pallas import tpu_sc as plsc

N = 10000
E = 320000
CH = 128
NEG = 0.2

NPAD = 10240
NG = 16
EPG = 20672
NCH2 = EPG // 32
ETOT = NG * EPG
TRASH = N


def _prep_body(x_ref, w_ref, emb_ref, att_ref, xh_ref, aij_ref):
    xb = x_ref[...]
    xh = jnp.dot(xb, w_ref[...], preferred_element_type=jnp.float32)
    xh_ref[...] = xh
    t1 = jnp.dot(xh, att_ref[...].T, preferred_element_type=jnp.float32)
    t2 = jnp.dot(emb_ref[...], att_ref[...].T, preferred_element_type=jnp.float32)
    ai = t1[:, 0] + t2[:, 2]
    aj = t1[:, 1] + t2[:, 3]
    z = jnp.zeros((14, ai.shape[0]), jnp.float32)
    aij_ref[...] = jnp.concatenate([ai[None, :], aj[None, :], z], axis=0)


def _tc_prep(x_pad, W, emb_pad, attv):
    blk = 1024
    return pl.pallas_call(
        _prep_body,
        grid=(NPAD // blk,),
        in_specs=[
            pl.BlockSpec((blk, CH), lambda i: (i, 0)),
            pl.BlockSpec((CH, CH), lambda i: (0, 0)),
            pl.BlockSpec((blk, CH), lambda i: (i, 0)),
            pl.BlockSpec((8, CH), lambda i: (0, 0)),
        ],
        out_specs=[
            pl.BlockSpec((blk, CH), lambda i: (i, 0)),
            pl.BlockSpec((16, blk), lambda i: (0, i)),
        ],
        out_shape=[
            jax.ShapeDtypeStruct((NPAD, CH), jnp.float32),
            jax.ShapeDtypeStruct((16, NPAD), jnp.float32),
        ],
    )(x_pad, W, emb_pad, attv)


def _post_body(acc_ref, den_ref, bias_ref, gamma_ref, beta_ref, out_ref):
    a = jnp.concatenate([acc_ref[q, :N, :] for q in range(4)], axis=1)
    den = den_ref[0, :N]
    pre = a / (den[:, None] + 1e-16) + bias_ref[...]
    mean = jnp.mean(pre, axis=0, keepdims=True)
    dlt = pre - mean
    var = jnp.mean(dlt * dlt, axis=0, keepdims=True)
    o = dlt * (gamma_ref[...] / jnp.sqrt(var + 1e-5)) + beta_ref[...]
    out_ref[...] = jnp.maximum(o, 0.0)


def _tc_post(acc, dens, bias2, gamma2, beta2):
    return pl.pallas_call(
        _post_body,
        out_shape=jax.ShapeDtypeStruct((N, CH), jnp.float32),
    )(acc, dens, bias2, gamma2, beta2)


def _make_sc_edges():
    mesh = plsc.VectorSubcoreMesh(core_axis_name="c", subcore_axis_name="s")

    @functools.partial(
        pl.kernel,
        mesh=mesh,
        compiler_params=pltpu.CompilerParams(
            needs_layout_passes=False, use_tc_tiling_on_sc=False),
        out_type=[
            pltpu.HBM((4, NPAD, CH // 4), jnp.float32),
            pltpu.HBM((2, NPAD // 16, 16), jnp.float32),
        ],
        scratch_types=[
            pltpu.VMEM((NPAD,), jnp.float32),
            pltpu.VMEM((NPAD,), jnp.float32),
            pltpu.VMEM((NPAD // 16, 16), jnp.float32),
            pltpu.VMEM((NPAD // 16,), jnp.int32),
            pltpu.VMEM((NCH2, 32), jnp.int32),
            pltpu.VMEM((NCH2, 32), jnp.int32),
            pltpu.VMEM((NCH2, 32), jnp.float32),
            pltpu.VMEM((32, CH // 4), jnp.float32),
            pltpu.VMEM((32, CH // 4), jnp.float32),
            pltpu.VMEM((32, CH // 4), jnp.float32),
            pltpu.VMEM((32, CH // 4), jnp.float32),
            pltpu.VMEM((128, CH // 4), jnp.float32),
            pltpu.VMEM_SHARED((NPAD, CH // 4), jnp.float32),
            pltpu.VMEM_SHARED((NPAD // 16, 16), jnp.float32),
            pltpu.SemaphoreType.DMA,
            pltpu.SemaphoreType.DMA,
            pltpu.SemaphoreType.DMA,
            pltpu.SemaphoreType.DMA,
        ],
    )
    def sc_edges(xh_hbm, src_hbm, dst_hbm, aij_hbm, acc_out, den_out,
                 ai_v, aj_v, den_v, blk_v, src_v, dst_v, p_v,
                 rows_a, rows_b, scl_a, scl_b, zer_v, acc_sh, den_sh,
                 gsem_a, gsem_b, ssem_a, ssem_b):
        c = lax.axis_index("c")
        s = lax.axis_index("s")

        pltpu.sync_copy(aij_hbm.at[0], ai_v)
        pltpu.sync_copy(aij_hbm.at[1], aj_v)
        pltpu.sync_copy(src_hbm.at[s], src_v)
        pltpu.sync_copy(dst_hbm.at[s], dst_v)

        z16 = jnp.zeros((16,), jnp.float32)
        iota16 = lax.iota(jnp.int32, 16)

        def zden(i, carry):
            den_v[i, :] = z16
            blk_v[pl.ds(i * 16, 16)] = iota16 + i * 16
            return carry
        lax.fori_loop(0, NPAD // 256, zden, 0)

        def zden2(i, carry):
            den_v[i, :] = z16
            return carry
        lax.fori_loop(NPAD // 256, NPAD // 16, zden2, 0)

        def zzer(i, carry):
            for k in range(2):
                zer_v[i, pl.ds(k * 16, 16)] = z16
            return carry
        lax.fori_loop(0, 128, zzer, 0)

        @pl.when(s == 0)
        def _():
            pltpu.sync_copy(den_v, den_sh)
        plsc.subcore_barrier()

        def ph1(g, carry):
            for h in (0, 16):
                src16 = src_v[g, pl.ds(h, 16)]
                dst16 = dst_v[g, pl.ds(h, 16)]
                ai16 = plsc.load_gather(ai_v, [dst16])
                aj16 = plsc.load_gather(aj_v, [src16])
                al = ai16 + aj16
                al = jnp.where(al >= 0.0, al, NEG * al)
                p16 = jnp.exp(al)
                p_v[g, pl.ds(h, 16)] = p16
                plsc.addupdate_scatter(
                    den_v, [lax.shift_right_logical(dst16, 4), dst16 & 15],
                    p16)
            return carry
        lax.fori_loop(0, NCH2, ph1, 0)

        pltpu.sync_copy(den_v, den_sh.at[blk_v], add=True)

        def shift_src(delta):
            def body(g, carry):
                for h in (0, 16):
                    src_v[g, pl.ds(h, 16)] = src_v[g, pl.ds(h, 16)] + delta
                return carry
            lax.fori_loop(0, NCH2, body, 0)

        def gather(j, rows, gsem):
            pltpu.async_copy(xh_hbm.at[src_v.at[j]], rows, gsem)

        def gwait(j, rows, gsem):
            pltpu.make_async_copy(xh_hbm.at[src_v.at[j]], rows, gsem).wait()

        def scatter(j, scl, ssem):
            pltpu.async_copy(scl, acc_sh.at[dst_v.at[j]], ssem, add=True)

        def swait(j, scl, ssem):
            pltpu.make_async_copy(
                scl, acc_sh.at[dst_v.at[j]], ssem).wait()

        def scale(j, rows, scl):
            jj = jnp.zeros((16,), jnp.int32) + j
            for r in range(32):
                rr = jnp.full((16,), r, jnp.int32)
                pv = plsc.load_gather(p_v, [jj, rr])
                for k in (0, 16):
                    scl[r, pl.ds(k, 16)] = rows[r, pl.ds(k, 16)] * pv

        for q in range(2):
            shift_src(c * NPAD if q == 0 else 2 * NPAD)

            def zacc(m, carry):
                pltpu.sync_copy(zer_v, acc_sh.at[pl.ds(s * 640 + m * 128, 128)])
                return carry
            lax.fori_loop(0, 5, zacc, 0)
            plsc.subcore_barrier()

            gather(0, rows_a, gsem_a)
            gather(1, rows_b, gsem_b)

            def ph2(go, carry):
                j0 = 2 * go
                j1 = j0 + 1
                gwait(j0, rows_a, gsem_a)

                @pl.when(go > 0)
                def _():
                    swait(j0 - 2, scl_a, ssem_a)
                scale(j0, rows_a, scl_a)
                scatter(j0, scl_a, ssem_a)
                gather(j0 + 2, rows_a, gsem_a)

                gwait(j1, rows_b, gsem_b)

                @pl.when(go > 0)
                def _():
                    swait(j1 - 2, scl_b, ssem_b)
                scale(j1, rows_b, scl_b)
                scatter(j1, scl_b, ssem_b)
                gather(j1 + 2, rows_b, gsem_b)
                return carry
            lax.fori_loop(0, NCH2 // 2 - 1, ph2, 0)

            jl0 = NCH2 - 2
            jl1 = NCH2 - 1
            gwait(jl0, rows_a, gsem_a)
            swait(jl0 - 2, scl_a, ssem_a)
            scale(jl0, rows_a, scl_a)
            scatter(jl0, scl_a, ssem_a)
            gwait(jl1, rows_b, gsem_b)
            swait(jl1 - 2, scl_b, ssem_b)
            scale(jl1, rows_b, scl_b)
            scatter(jl1, scl_b, ssem_b)
            swait(jl0, scl_a, ssem_a)
            swait(jl1, scl_b, ssem_b)
            plsc.subcore_barrier()

            @pl.when(s == 0)
            def _():
                pltpu.sync_copy(acc_sh, acc_out.at[c + 2 * q])
            plsc.subcore_barrier()

        @pl.when(s == 0)
        def _():
            pltpu.sync_copy(den_sh, den_out.at[c])

    return sc_edges


def kernel(x, edge_index, embedding, W, att_i, att_j, att_em_i, att_em_j,
           bias, gamma, beta):
    src = edge_index[0]
    dst = edge_index[1]
    dst = jnp.where(src == dst, TRASH, dst)
    loop = jnp.arange(N, dtype=jnp.int32)
    pad = ETOT - (E + N)
    src_all = jnp.concatenate([src, loop, jnp.zeros((pad,), jnp.int32)])
    dst_all = jnp.concatenate([dst, loop, jnp.full((pad,), TRASH, jnp.int32)])
    src3 = src_all.reshape(NG, NCH2, 32)
    dst3 = dst_all.reshape(NG, NCH2, 32)

    x_pad = jnp.pad(x, ((0, NPAD - N), (0, 0)))
    emb_pad = jnp.pad(embedding, ((0, NPAD - N), (0, 0)))
    attv = jnp.concatenate(
        [att_i[0], att_j[0], att_em_i[0], att_em_j[0],
         jnp.zeros((4, CH), jnp.float32)], axis=0)

    xh, aij = _tc_prep(x_pad, W, emb_pad, attv)
    xh2 = xh.reshape(NPAD, 4, CH // 4).transpose(1, 0, 2).reshape(4 * NPAD, CH // 4)

    sc_edges = _make_sc_edges()
    acc, dens = sc_edges(xh2, src3, dst3, aij)

    return _tc_post(acc, dens.reshape(2, NPAD),
                    bias[None, :], gamma[None, :], beta[None, :])
